# Initial kernel scaffold; baseline (speedup 1.0000x reference)
import jax
import jax.numpy as jnp
from jax.experimental import pallas as pl


def kernel(x, edge_index, affine, W, b):
    raise NotImplementedError("write your pallas kernel here")


# speedup vs baseline: 3.6233x; 3.6233x over previous
import functools

import jax
import jax.numpy as jnp
from jax import lax
from jax.experimental import pallas as pl
from jax.experimental.pallas import tpu as pltpu
from jax.experimental.pallas import tpu_sc as plsc

N = 10000
D = 128
OUT = 128
E = 320000

NC = 2
NS = 16
NW = NC * NS
LANES = 16

CHUNK = 128
CHUNKS_PER_TILE = -(-E // (NW * CHUNK))
E_PAD = NW * CHUNKS_PER_TILE * CHUNK
N_PAD = 10240
ROWS_PER_TILE = N_PAD // NS
ZROWS = 128


def _sc_body(src_hbm, dst_hbm, aff_hbm, x_hbm, out_hbm,
             agg_sh, zbuf, src_v, dst_v, aff_v, rows_v, sem):
    cid = lax.axis_index("c")
    sid = lax.axis_index("s")
    wid = sid * NC + cid

    @pl.loop(0, ZROWS)
    def _zero_rows(r):
        for j in range(D // LANES):
            zbuf[r, pl.ds(j * LANES, LANES)] = jnp.zeros((LANES,), jnp.float32)

    for k in range(ROWS_PER_TILE // ZROWS):
        pltpu.sync_copy(zbuf, agg_sh.at[pl.ds(sid * ROWS_PER_TILE + k * ZROWS, ZROWS)])

    plsc.subcore_barrier()

    base = wid * CHUNKS_PER_TILE * CHUNK

    @pl.loop(0, CHUNKS_PER_TILE)
    def _chunk(g):
        off = base + g * CHUNK
        pltpu.sync_copy(src_hbm.at[pl.ds(off, CHUNK)], src_v)
        pltpu.sync_copy(dst_hbm.at[pl.ds(off, CHUNK)], dst_v)
        pltpu.sync_copy(aff_hbm.at[pl.ds(off, CHUNK)], aff_v)
        pltpu.async_copy(x_hbm.at[src_v], rows_v, sem).wait()

        @pl.loop(0, CHUNK // LANES)
        def _scale(e16):
            avec = aff_v[pl.ds(e16 * LANES, LANES)]
            for l in range(LANES):
                a = avec[l]
                e = e16 * LANES + l
                for j in range(D // LANES):
                    sl = pl.ds(j * LANES, LANES)
                    rows_v[e, sl] = rows_v[e, sl] * a

        pltpu.sync_copy(rows_v, agg_sh.at[dst_v], add=True)

    plsc.subcore_barrier()

    r0 = sid * ROWS_PER_TILE
    pltpu.sync_copy(agg_sh.at[pl.ds(r0, ROWS_PER_TILE)],
                    out_hbm.at[cid, pl.ds(r0, ROWS_PER_TILE)])


@jax.jit
def _segment_sum_sc(src, dst, aff, x):
    mesh = plsc.VectorSubcoreMesh(core_axis_name="c", subcore_axis_name="s")
    return pl.kernel(
        _sc_body,
        out_type=jax.ShapeDtypeStruct((NC, N_PAD, D), jnp.float32),
        mesh=mesh,
        scratch_types=[
            pltpu.VMEM_SHARED((N_PAD, D), jnp.float32),
            pltpu.VMEM((ZROWS, D), jnp.float32),
            pltpu.VMEM((CHUNK,), jnp.int32),
            pltpu.VMEM((CHUNK,), jnp.int32),
            pltpu.VMEM((CHUNK,), jnp.float32),
            pltpu.VMEM((CHUNK, D), jnp.float32),
            pltpu.SemaphoreType.DMA,
        ],
    )(src, dst, aff, x)


ROW_BLK = 1000


def _mm_body(x_ref, p0_ref, p1_ref, w1_ref, w2_ref, b_ref, o_ref):
    agg = p0_ref[...] + p1_ref[...]
    acc = jnp.dot(x_ref[...], w1_ref[...], preferred_element_type=jnp.float32)
    acc = acc + jnp.dot(agg, w2_ref[...], preferred_element_type=jnp.float32)
    o_ref[...] = acc + b_ref[...]


@jax.jit
def _concat_linear_tc(x, p0, p1, w1t, w2t, b2d):
    grid = (N // ROW_BLK,)
    return pl.pallas_call(
        _mm_body,
        grid=grid,
        in_specs=[
            pl.BlockSpec((ROW_BLK, D), lambda i: (i, 0)),
            pl.BlockSpec((ROW_BLK, D), lambda i: (i, 0)),
            pl.BlockSpec((ROW_BLK, D), lambda i: (i, 0)),
            pl.BlockSpec((D, OUT), lambda i: (0, 0)),
            pl.BlockSpec((D, OUT), lambda i: (0, 0)),
            pl.BlockSpec((1, OUT), lambda i: (0, 0)),
        ],
        out_specs=pl.BlockSpec((ROW_BLK, OUT), lambda i: (i, 0)),
        out_shape=jax.ShapeDtypeStruct((N, OUT), jnp.float32),
    )(x, p0, p1, w1t, w2t, b2d)


def kernel(x, edge_index, affine, W, b):
    pad = E_PAD - E
    src = jnp.concatenate([edge_index[0], jnp.zeros((pad,), jnp.int32)])
    dst = jnp.concatenate([edge_index[1], jnp.zeros((pad,), jnp.int32)])
    aff = jnp.concatenate([affine, jnp.zeros((pad,), jnp.float32)])

    partials = _segment_sum_sc(src, dst, aff, x)

    w1t = W[:, :D].T
    w2t = W[:, D:].T
    b2d = b.reshape(1, OUT)
    return _concat_linear_tc(x, partials[0], partials[1], w1t, w2t, b2d)
